# Initial kernel scaffold; baseline (speedup 1.0000x reference)
import jax
import jax.numpy as jnp
from jax.experimental import pallas as pl


def kernel(x, edge_index, perm):
    raise NotImplementedError("write your pallas kernel here")


# speedup vs baseline: 154.8179x; 154.8179x over previous
import functools

import jax
import jax.numpy as jnp
from jax import lax
from jax.experimental import pallas as pl
from jax.experimental.pallas import tpu as pltpu
from jax.experimental.pallas import tpu_sc as plsc

N_NODES = 10000
D_FEAT = 128
N_EDGE_ELEMS = 640000

NC = 2
NS = 16
NW = NC * NS
L = 16

ROWS_PER_W = 320
X_CHUNK = 80
N_XCHUNKS = ROWS_PER_W // X_CHUNK
EDGES_PER_W = N_EDGE_ELEMS // NW


def _sc_body(x_hbm, edge_hbm, perm_hbm, out_x_hbm, out_e_hbm,
             perm_v, inv_v, idx_v, rows_v, edge_v,
             sem_perm, sem_idx, sem_edge, sem_rows):
    c = lax.axis_index("c")
    s = lax.axis_index("s")
    wid = s * NC + c

    cp_perm = pltpu.async_copy(perm_hbm, perm_v, sem_perm)

    ebase = wid * EDGES_PER_W
    cp_edge = pltpu.async_copy(
        edge_hbm.at[pl.ds(ebase, EDGES_PER_W)], edge_v, sem_edge)

    xbase = jnp.minimum(wid * ROWS_PER_W, N_NODES - ROWS_PER_W)
    idx_cps = []
    for j in range(N_XCHUNKS):
        idx_cps.append(pltpu.async_copy(
            perm_hbm.at[pl.ds(xbase + j * X_CHUNK, X_CHUNK)],
            idx_v.at[j], sem_idx))
    for cp in idx_cps:
        cp.wait()

    row_cps = []
    for j in range(N_XCHUNKS):
        row_cps.append(pltpu.async_copy(
            x_hbm.at[idx_v.at[j]],
            rows_v.at[pl.ds(j * X_CHUNK, X_CHUNK)], sem_rows))

    cp_perm.wait()

    def inv_body(i, _):
        p = perm_v[pl.ds(i * L, L)]
        vals = lax.iota(jnp.int32, L) + i * L
        plsc.store_scatter(inv_v, [p], vals)
        return _

    lax.fori_loop(0, N_NODES // L, inv_body, None)

    cp_edge.wait()

    def edge_body(i, _):
        e = edge_v[pl.ds(i * L, L)]
        edge_v[pl.ds(i * L, L)] = plsc.load_gather(inv_v, [e])
        return _

    lax.fori_loop(0, EDGES_PER_W // L, edge_body, None)

    pltpu.sync_copy(edge_v, out_e_hbm.at[pl.ds(ebase, EDGES_PER_W)])

    for cp in row_cps:
        cp.wait()
    pltpu.sync_copy(rows_v, out_x_hbm.at[pl.ds(xbase, ROWS_PER_W)])


@jax.jit
def kernel(x, edge_index, perm):
    edge_flat = edge_index.astype(jnp.int32).reshape(-1)
    perm32 = perm.astype(jnp.int32)

    run = pl.kernel(
        _sc_body,
        out_type=(
            jax.ShapeDtypeStruct((N_NODES, D_FEAT), jnp.float32),
            jax.ShapeDtypeStruct((N_EDGE_ELEMS,), jnp.int32),
        ),
        mesh=plsc.VectorSubcoreMesh(
            core_axis_name="c", subcore_axis_name="s"),
        compiler_params=pltpu.CompilerParams(needs_layout_passes=False),
        scratch_types=[
            pltpu.VMEM((N_NODES,), jnp.int32),
            pltpu.VMEM((N_NODES,), jnp.int32),
            pltpu.VMEM((N_XCHUNKS, X_CHUNK), jnp.int32),
            pltpu.VMEM((ROWS_PER_W, D_FEAT), jnp.float32),
            pltpu.VMEM((EDGES_PER_W,), jnp.int32),
            pltpu.SemaphoreType.DMA,
            pltpu.SemaphoreType.DMA,
            pltpu.SemaphoreType.DMA,
            pltpu.SemaphoreType.DMA,
        ],
    )
    new_x, new_edge_flat = run(x, edge_flat, perm32)
    return new_x, new_edge_flat.reshape(2, N_EDGE_ELEMS // 2)


# speedup vs baseline: 212.4832x; 1.3725x over previous
import functools

import jax
import jax.numpy as jnp
from jax import lax
from jax.experimental import pallas as pl
from jax.experimental.pallas import tpu as pltpu
from jax.experimental.pallas import tpu_sc as plsc

N_NODES = 10000
D_FEAT = 128
N_EDGE_ELEMS = 640000

NC = 2
NS = 16
NW = NC * NS
L = 16

ROWS_PER_W = 320
X_CHUNK = 80
N_XCHUNKS = ROWS_PER_W // X_CHUNK
EDGES_PER_W = N_EDGE_ELEMS // NW


def _sc_body(x_hbm, edge_hbm, perm_hbm, out_x_hbm, out_e_hbm,
             perm_v, inv_v, idx_v, rows_v, edge_v, eout_v,
             sem_perm, sem_idx, sem_edge, sem_rows):
    c = lax.axis_index("c")
    s = lax.axis_index("s")
    wid = s * NC + c

    cp_perm = pltpu.async_copy(perm_hbm, perm_v, sem_perm)

    ebase = wid * EDGES_PER_W
    cp_edge = pltpu.async_copy(
        edge_hbm.at[pl.ds(ebase, EDGES_PER_W)], edge_v, sem_edge)

    xbase = jnp.minimum(wid * ROWS_PER_W, N_NODES - ROWS_PER_W)
    idx_cps = []
    for j in range(N_XCHUNKS):
        idx_cps.append(pltpu.async_copy(
            perm_hbm.at[pl.ds(xbase + j * X_CHUNK, X_CHUNK)],
            idx_v.at[j], sem_idx))
    for cp in idx_cps:
        cp.wait()

    row_cps = []
    for j in range(N_XCHUNKS):
        row_cps.append(pltpu.async_copy(
            x_hbm.at[idx_v.at[j]],
            rows_v.at[pl.ds(j * X_CHUNK, X_CHUNK)], sem_rows))

    cp_perm.wait()

    @plsc.parallel_loop(0, N_NODES // L, unroll=8)
    def _inv_loop(i):
        p = perm_v[pl.ds(i * L, L)]
        plsc.store_scatter(inv_v, [p], lax.iota(jnp.int32, L) + i * L)

    cp_edge.wait()

    @plsc.parallel_loop(0, EDGES_PER_W // L, unroll=8)
    def _edge_loop(i):
        e = edge_v[pl.ds(i * L, L)]
        eout_v[pl.ds(i * L, L)] = plsc.load_gather(inv_v, [e])

    pltpu.sync_copy(eout_v, out_e_hbm.at[pl.ds(ebase, EDGES_PER_W)])

    for cp in row_cps:
        cp.wait()
    pltpu.sync_copy(rows_v, out_x_hbm.at[pl.ds(xbase, ROWS_PER_W)])


@jax.jit
def kernel(x, edge_index, perm):
    edge_flat = edge_index.astype(jnp.int32).reshape(-1)
    perm32 = perm.astype(jnp.int32)

    run = pl.kernel(
        _sc_body,
        out_type=(
            jax.ShapeDtypeStruct((N_NODES, D_FEAT), jnp.float32),
            jax.ShapeDtypeStruct((N_EDGE_ELEMS,), jnp.int32),
        ),
        mesh=plsc.VectorSubcoreMesh(
            core_axis_name="c", subcore_axis_name="s"),
        compiler_params=pltpu.CompilerParams(needs_layout_passes=False),
        scratch_types=[
            pltpu.VMEM((N_NODES,), jnp.int32),
            pltpu.VMEM((N_NODES,), jnp.int32),
            pltpu.VMEM((N_XCHUNKS, X_CHUNK), jnp.int32),
            pltpu.VMEM((ROWS_PER_W, D_FEAT), jnp.float32),
            pltpu.VMEM((EDGES_PER_W,), jnp.int32),
            pltpu.VMEM((EDGES_PER_W,), jnp.int32),
            pltpu.SemaphoreType.DMA,
            pltpu.SemaphoreType.DMA,
            pltpu.SemaphoreType.DMA,
            pltpu.SemaphoreType.DMA,
        ],
    )
    new_x, new_edge_flat = run(x, edge_flat, perm32)
    return new_x, new_edge_flat.reshape(2, N_EDGE_ELEMS // 2)
